# baseline (speedup 1.0000x reference)
import jax
import jax.numpy as jnp
from jax import lax
from jax.experimental import pallas as pl
from jax.experimental.pallas import tpu as pltpu
from jax.experimental.pallas import tpu_sc as plsc

_N = 10000
_E = 160000
_D = 256
_H = 256

_NS = 16
_K = 128
_EP = 163840
_CR = _EP // _K
_CT = _CR // _NS
_SCH = 16
_SUP = _CT // _SCH
_NB = 2
_NP = 10112
_RT = _NP // _NS
_TRASH = 10100

_BN = 1000
_GRID = _N // _BN

_mesh = plsc.VectorSubcoreMesh(core_axis_name="c", subcore_axis_name="s",
                               num_cores=1)

_SCRATCH = (
    [pltpu.VMEM((_SCH, _K), jnp.int32)] * 2
    + [pltpu.VMEM((_SCH, _K), jnp.int32)] * 2
    + [pltpu.VMEM((_K, 128), jnp.float32)] * _NB
    + [pltpu.VMEM_SHARED((_NP, 128), jnp.float32)]
    + [pltpu.SemaphoreType.DMA] * 2
    + [pltpu.SemaphoreType.DMA] * _NB
)


def _split_scratch(rest):
    return (rest[0:2], rest[2:4], rest[4:4 + _NB], rest[4 + _NB],
            rest[5 + _NB:7 + _NB], rest[7 + _NB:])


def _emit_count(dst_h, acc, dblk, ones_row, semI, row0):
    pltpu.sync_copy(dst_h.at[pl.ds(row0, _SCH)], dblk[0])
    for si in range(_SUP):
        p = si % 2
        q = 1 - p
        if si + 1 < _SUP:
            nxt = pl.ds(row0 + (si + 1) * _SCH, _SCH)
            pltpu.async_copy(dst_h.at[nxt], dblk[q], semI[q])

        def mid(r, carry, p=p):
            pltpu.sync_copy(ones_row, acc.at[dblk[p].at[r]], add=True)
            return carry

        lax.fori_loop(0, _SCH, mid, 0)
        if si + 1 < _SUP:
            pltpu.make_async_copy(dst_h.at[nxt], dblk[q], semI[q]).wait()


def _emit_agg(x_h, src_h, dst_h, acc, sblk, dblk, rows, semI, semG, row0):
    pltpu.sync_copy(src_h.at[pl.ds(row0, _SCH)], sblk[0])
    pltpu.sync_copy(dst_h.at[pl.ds(row0, _SCH)], dblk[0])
    for si in range(_SUP):
        p = si % 2
        q = 1 - p
        if si + 1 < _SUP:
            nxt = pl.ds(row0 + (si + 1) * _SCH, _SCH)
            pltpu.async_copy(src_h.at[nxt], sblk[q], semI[q])
            pltpu.async_copy(dst_h.at[nxt], dblk[q], semI[q])

        for b in range(_NB):
            pltpu.async_copy(x_h.at[sblk[p].at[b]], rows[b], semG[b])

        def mid(r, carry, p=p):
            for b in range(_NB):
                j = r * _NB + b
                pltpu.make_async_copy(x_h.at[sblk[p].at[j]], rows[b],
                                      semG[b]).wait()
                pltpu.sync_copy(rows[b], acc.at[dblk[p].at[j]], add=True)
                pltpu.async_copy(x_h.at[sblk[p].at[j + _NB]], rows[b],
                                 semG[b])
            return carry

        lax.fori_loop(0, _SCH // _NB - 1, mid, 0)

        for b in range(_NB):
            j = _SCH - _NB + b
            pltpu.make_async_copy(x_h.at[sblk[p].at[j]], rows[b],
                                  semG[b]).wait()
            pltpu.sync_copy(rows[b], acc.at[dblk[p].at[j]], add=True)

        if si + 1 < _SUP:
            pltpu.make_async_copy(src_h.at[nxt], sblk[q], semI[q]).wait()
            pltpu.make_async_copy(dst_h.at[nxt], dblk[q], semI[q]).wait()


def _conv1_body(xL_h, xR_h, src_h, dst_h, zrow_h, outC, outL, outR, *rest):
    sblk, dblk, rows, acc, semI, semG = _split_scratch(rest)
    s = lax.axis_index("s")
    rs = pl.ds(s * _RT, _RT)
    row0 = s * _CT

    def fill(r, carry):
        for l in range(8):
            rows[0][r, pl.ds(l * 16, 16)] = jnp.ones((16,), jnp.float32)
        return carry

    lax.fori_loop(0, _K, fill, 0)
    pltpu.sync_copy(zrow_h, acc.at[rs])
    plsc.subcore_barrier()
    _emit_count(dst_h, acc, dblk, rows[0], semI, row0)
    plsc.subcore_barrier()
    pltpu.sync_copy(acc.at[rs], outC.at[rs])
    pltpu.sync_copy(zrow_h, acc.at[rs])
    plsc.subcore_barrier()

    _emit_agg(xL_h, src_h, dst_h, acc, sblk, dblk, rows, semI, semG, row0)
    plsc.subcore_barrier()
    pltpu.sync_copy(acc.at[rs], outL.at[rs])
    pltpu.sync_copy(zrow_h, acc.at[rs])
    plsc.subcore_barrier()

    _emit_agg(xR_h, src_h, dst_h, acc, sblk, dblk, rows, semI, semG, row0)
    plsc.subcore_barrier()
    pltpu.sync_copy(acc.at[rs], outR.at[rs])


_conv1 = pl.kernel(
    _conv1_body,
    out_type=[jax.ShapeDtypeStruct((_NP, 128), jnp.float32)] * 3,
    mesh=_mesh,
    scratch_types=_SCRATCH,
)


def _conv2_body(xL_h, xR_h, src_h, dst_h, zrow_h, outL, outR, *rest):
    sblk, dblk, rows, acc, semI, semG = _split_scratch(rest)
    s = lax.axis_index("s")
    rs = pl.ds(s * _RT, _RT)
    row0 = s * _CT

    pltpu.sync_copy(zrow_h, acc.at[rs])
    plsc.subcore_barrier()
    _emit_agg(xL_h, src_h, dst_h, acc, sblk, dblk, rows, semI, semG, row0)
    plsc.subcore_barrier()
    pltpu.sync_copy(acc.at[rs], outL.at[rs])
    pltpu.sync_copy(zrow_h, acc.at[rs])
    plsc.subcore_barrier()

    _emit_agg(xR_h, src_h, dst_h, acc, sblk, dblk, rows, semI, semG, row0)
    plsc.subcore_barrier()
    pltpu.sync_copy(acc.at[rs], outR.at[rs])


_conv2 = pl.kernel(
    _conv2_body,
    out_type=[jax.ShapeDtypeStruct((_NP, 128), jnp.float32)] * 2,
    mesh=_mesh,
    scratch_types=_SCRATCH,
)


def _dot_t(a, w):
    return lax.dot_general(a, w, (((1,), (1,)), ((), ())),
                           preferred_element_type=jnp.float32)


def _dense1_body(aggL_ref, aggR_ref, cnt_ref, x_ref, wl_ref, bl_ref, wr_ref,
                 hL_ref, hR_ref, h_ref):
    inv = 1.0 / jnp.maximum(cnt_ref[:, 0:1], 1.0)
    agg = jnp.concatenate([aggL_ref[:, :] * inv, aggR_ref[:, :] * inv], axis=1)
    h = _dot_t(agg, wl_ref[:, :]) + bl_ref[:, :] + _dot_t(x_ref[:, :], wr_ref[:, :])
    h = jnp.maximum(h, 0.0)
    h_ref[:, :] = h
    hL_ref[:, :] = h[:, :128]
    hR_ref[:, :] = h[:, 128:]


_dense1 = pl.pallas_call(
    _dense1_body,
    grid=(_GRID,),
    in_specs=[
        pl.BlockSpec((_BN, 128), lambda i: (i, 0)),
        pl.BlockSpec((_BN, 128), lambda i: (i, 0)),
        pl.BlockSpec((_BN, 128), lambda i: (i, 0)),
        pl.BlockSpec((_BN, _D), lambda i: (i, 0)),
        pl.BlockSpec((_H, _D), lambda i: (0, 0)),
        pl.BlockSpec((1, _H), lambda i: (0, 0)),
        pl.BlockSpec((_H, _D), lambda i: (0, 0)),
    ],
    out_specs=[
        pl.BlockSpec((_BN, 128), lambda i: (i, 0)),
        pl.BlockSpec((_BN, 128), lambda i: (i, 0)),
        pl.BlockSpec((_BN, _H), lambda i: (i, 0)),
    ],
    out_shape=[
        jax.ShapeDtypeStruct((_N, 128), jnp.float32),
        jax.ShapeDtypeStruct((_N, 128), jnp.float32),
        jax.ShapeDtypeStruct((_N, _H), jnp.float32),
    ],
)


def _dense2_body(aggL_ref, aggR_ref, cnt_ref, h1_ref, w2l_ref, b2l_ref,
                 w2r_ref, wl1_ref, bl1_ref, wl2_ref, bl2_ref, out_ref):
    inv = 1.0 / jnp.maximum(cnt_ref[:, 0:1], 1.0)
    agg = jnp.concatenate([aggL_ref[:, :] * inv, aggR_ref[:, :] * inv], axis=1)
    h = _dot_t(agg, w2l_ref[:, :]) + b2l_ref[:, :] + _dot_t(h1_ref[:, :], w2r_ref[:, :])
    h = jnp.maximum(h, 0.0)
    h = _dot_t(h, wl1_ref[:, :]) + bl1_ref[:, :]
    h = jnp.maximum(h, 0.0)
    lg = jnp.sum(h * wl2_ref[0:1, :], axis=1, keepdims=True)
    out_ref[:, :] = lg + bl2_ref[0, 0]


_dense2 = pl.pallas_call(
    _dense2_body,
    grid=(_GRID,),
    in_specs=[
        pl.BlockSpec((_BN, 128), lambda i: (i, 0)),
        pl.BlockSpec((_BN, 128), lambda i: (i, 0)),
        pl.BlockSpec((_BN, 128), lambda i: (i, 0)),
        pl.BlockSpec((_BN, _H), lambda i: (i, 0)),
        pl.BlockSpec((_H, _H), lambda i: (0, 0)),
        pl.BlockSpec((1, _H), lambda i: (0, 0)),
        pl.BlockSpec((_H, _H), lambda i: (0, 0)),
        pl.BlockSpec((_H, _H), lambda i: (0, 0)),
        pl.BlockSpec((1, _H), lambda i: (0, 0)),
        pl.BlockSpec((1, _H), lambda i: (0, 0)),
        pl.BlockSpec((1, 1), lambda i: (0, 0)),
    ],
    out_specs=pl.BlockSpec((_BN, 1), lambda i: (i, 0)),
    out_shape=jax.ShapeDtypeStruct((_N, 1), jnp.float32),
)


def kernel(x, edge_index, W1l, b1l, W1r, W2l, b2l, W2r, Wlin1, blin1, Wlin2,
           blin2):
    npad = _EP - _E
    src2d = jnp.concatenate(
        [edge_index[0], jnp.zeros((npad,), jnp.int32)]).reshape(_CR, _K)
    dst2d = jnp.concatenate(
        [edge_index[1], jnp.full((npad,), _TRASH, jnp.int32)]).reshape(_CR, _K)
    xL = x[:, :128]
    xR = x[:, 128:]
    zrow = jnp.zeros((_RT, 128), jnp.float32)

    cntc, aL, aR = _conv1(xL, xR, src2d, dst2d, zrow)
    h1L, h1R, h1 = _dense1(aL, aR, cntc, x, W1l, b1l.reshape(1, _H), W1r)
    a2L, a2R = _conv2(h1L, h1R, src2d, dst2d, zrow)
    lg = _dense2(a2L, a2R, cntc, h1, W2l, b2l.reshape(1, _H), W2r,
                 Wlin1, blin1.reshape(1, _H), Wlin2, blin2.reshape(1, 1))
    return lg[:, 0]


# speedup vs baseline: 1.0382x; 1.0382x over previous
import jax
import jax.numpy as jnp
from jax import lax
from jax.experimental import pallas as pl
from jax.experimental.pallas import tpu as pltpu
from jax.experimental.pallas import tpu_sc as plsc

_N = 10000
_E = 160000
_D = 256
_H = 256

_NS = 16
_K = 128
_EP = 163840
_CR = _EP // _K
_CT = _CR // _NS
_SCH = 16
_SUP = _CT // _SCH
_NB = 2
_NP = 10112
_RT = _NP // _NS
_TRASH = 10100

_BN = 1000
_GRID = _N // _BN

_mesh = plsc.VectorSubcoreMesh(core_axis_name="c", subcore_axis_name="s",
                               num_cores=1)


def _agg_body(x_h, src_h, dst_h, zrow_h, out_h, *rest):
    sblk = rest[0:2]
    dblk = rest[2:4]
    rows = rest[4:4 + _NB]
    acc = rest[4 + _NB]
    semI = rest[5 + _NB:7 + _NB]
    semG = rest[7 + _NB:]
    s = lax.axis_index("s")
    rs = pl.ds(s * _RT, _RT)
    pltpu.sync_copy(zrow_h, acc.at[rs])
    row0 = s * _CT

    pltpu.sync_copy(src_h.at[pl.ds(row0, _SCH)], sblk[0])
    pltpu.sync_copy(dst_h.at[pl.ds(row0, _SCH)], dblk[0])
    plsc.subcore_barrier()

    for si in range(_SUP):
        p = si % 2
        q = 1 - p
        if si + 1 < _SUP:
            nxt = pl.ds(row0 + (si + 1) * _SCH, _SCH)
            pltpu.async_copy(src_h.at[nxt], sblk[q], semI[q])
            pltpu.async_copy(dst_h.at[nxt], dblk[q], semI[q])

        for b in range(_NB):
            pltpu.async_copy(x_h.at[sblk[p].at[b]], rows[b], semG[b])

        def mid(r, carry, p=p):
            for b in range(_NB):
                j = r * _NB + b
                pltpu.make_async_copy(x_h.at[sblk[p].at[j]], rows[b],
                                      semG[b]).wait()
                pltpu.sync_copy(rows[b], acc.at[dblk[p].at[j]], add=True)
                pltpu.async_copy(x_h.at[sblk[p].at[j + _NB]], rows[b],
                                 semG[b])
            return carry

        lax.fori_loop(0, _SCH // _NB - 1, mid, 0)

        for b in range(_NB):
            j = _SCH - _NB + b
            pltpu.make_async_copy(x_h.at[sblk[p].at[j]], rows[b],
                                  semG[b]).wait()
            pltpu.sync_copy(rows[b], acc.at[dblk[p].at[j]], add=True)

        if si + 1 < _SUP:
            pltpu.make_async_copy(src_h.at[nxt], sblk[q], semI[q]).wait()
            pltpu.make_async_copy(dst_h.at[nxt], dblk[q], semI[q]).wait()

    plsc.subcore_barrier()
    pltpu.sync_copy(acc.at[rs], out_h.at[rs])


_agg = pl.kernel(
    _agg_body,
    out_type=[jax.ShapeDtypeStruct((_NP, 128), jnp.float32)],
    mesh=_mesh,
    scratch_types=(
        [pltpu.VMEM((_SCH, _K), jnp.int32)] * 2
        + [pltpu.VMEM((_SCH, _K), jnp.int32)] * 2
        + [pltpu.VMEM((_K, 128), jnp.float32)] * _NB
        + [pltpu.VMEM_SHARED((_NP, 128), jnp.float32)]
        + [pltpu.SemaphoreType.DMA] * 2
        + [pltpu.SemaphoreType.DMA] * _NB
    ),
)


def _count_body(dst_h, zrow_h, ones_h, out_h, *rest):
    dblk = rest[0:2]
    ones = rest[2]
    cnt = rest[3]
    semI = rest[4:6]
    s = lax.axis_index("s")
    rs = pl.ds(s * _RT, _RT)
    pltpu.sync_copy(zrow_h, cnt.at[rs])
    pltpu.sync_copy(ones_h, ones)
    row0 = s * _CT
    pltpu.sync_copy(dst_h.at[pl.ds(row0, _SCH)], dblk[0])
    plsc.subcore_barrier()

    for si in range(_SUP):
        p = si % 2
        q = 1 - p
        if si + 1 < _SUP:
            nxt = pl.ds(row0 + (si + 1) * _SCH, _SCH)
            pltpu.async_copy(dst_h.at[nxt], dblk[q], semI[q])

        def mid(r, carry, p=p):
            pltpu.sync_copy(ones, cnt.at[dblk[p].at[r]], add=True)
            return carry

        lax.fori_loop(0, _SCH, mid, 0)

        if si + 1 < _SUP:
            pltpu.make_async_copy(dst_h.at[nxt], dblk[q], semI[q]).wait()

    plsc.subcore_barrier()
    pltpu.sync_copy(cnt.at[rs], out_h.at[rs])


_count = pl.kernel(
    _count_body,
    out_type=[jax.ShapeDtypeStruct((_NP, 128), jnp.float32)],
    mesh=_mesh,
    scratch_types=(
        [pltpu.VMEM((_SCH, _K), jnp.int32)] * 2
        + [pltpu.VMEM((_K, 128), jnp.float32)]
        + [pltpu.VMEM_SHARED((_NP, 128), jnp.float32)]
        + [pltpu.SemaphoreType.DMA] * 2
    ),
)


def _dot_t(a, w):
    return lax.dot_general(a, w, (((1,), (1,)), ((), ())),
                           preferred_element_type=jnp.float32)


def _dense1_body(aggL_ref, aggR_ref, cnt_ref, x_ref, wl_ref, bl_ref, wr_ref,
                 hL_ref, hR_ref, h_ref):
    inv = 1.0 / jnp.maximum(cnt_ref[:, 0:1], 1.0)
    agg = jnp.concatenate([aggL_ref[:, :] * inv, aggR_ref[:, :] * inv], axis=1)
    h = _dot_t(agg, wl_ref[:, :]) + bl_ref[:, :] + _dot_t(x_ref[:, :], wr_ref[:, :])
    h = jnp.maximum(h, 0.0)
    h_ref[:, :] = h
    hL_ref[:, :] = h[:, :128]
    hR_ref[:, :] = h[:, 128:]


_dense1 = pl.pallas_call(
    _dense1_body,
    grid=(_GRID,),
    in_specs=[
        pl.BlockSpec((_BN, 128), lambda i: (i, 0)),
        pl.BlockSpec((_BN, 128), lambda i: (i, 0)),
        pl.BlockSpec((_BN, 128), lambda i: (i, 0)),
        pl.BlockSpec((_BN, _D), lambda i: (i, 0)),
        pl.BlockSpec((_H, _D), lambda i: (0, 0)),
        pl.BlockSpec((1, _H), lambda i: (0, 0)),
        pl.BlockSpec((_H, _D), lambda i: (0, 0)),
    ],
    out_specs=[
        pl.BlockSpec((_BN, 128), lambda i: (i, 0)),
        pl.BlockSpec((_BN, 128), lambda i: (i, 0)),
        pl.BlockSpec((_BN, _H), lambda i: (i, 0)),
    ],
    out_shape=[
        jax.ShapeDtypeStruct((_N, 128), jnp.float32),
        jax.ShapeDtypeStruct((_N, 128), jnp.float32),
        jax.ShapeDtypeStruct((_N, _H), jnp.float32),
    ],
)


def _dense2_body(aggL_ref, aggR_ref, cnt_ref, h1_ref, w2l_ref, b2l_ref,
                 w2r_ref, wl1_ref, bl1_ref, wl2_ref, bl2_ref, out_ref):
    inv = 1.0 / jnp.maximum(cnt_ref[:, 0:1], 1.0)
    agg = jnp.concatenate([aggL_ref[:, :] * inv, aggR_ref[:, :] * inv], axis=1)
    h = _dot_t(agg, w2l_ref[:, :]) + b2l_ref[:, :] + _dot_t(h1_ref[:, :], w2r_ref[:, :])
    h = jnp.maximum(h, 0.0)
    h = _dot_t(h, wl1_ref[:, :]) + bl1_ref[:, :]
    h = jnp.maximum(h, 0.0)
    lg = jnp.sum(h * wl2_ref[0:1, :], axis=1, keepdims=True)
    out_ref[:, :] = lg + bl2_ref[0, 0]


_dense2 = pl.pallas_call(
    _dense2_body,
    grid=(_GRID,),
    in_specs=[
        pl.BlockSpec((_BN, 128), lambda i: (i, 0)),
        pl.BlockSpec((_BN, 128), lambda i: (i, 0)),
        pl.BlockSpec((_BN, 128), lambda i: (i, 0)),
        pl.BlockSpec((_BN, _H), lambda i: (i, 0)),
        pl.BlockSpec((_H, _H), lambda i: (0, 0)),
        pl.BlockSpec((1, _H), lambda i: (0, 0)),
        pl.BlockSpec((_H, _H), lambda i: (0, 0)),
        pl.BlockSpec((_H, _H), lambda i: (0, 0)),
        pl.BlockSpec((1, _H), lambda i: (0, 0)),
        pl.BlockSpec((1, _H), lambda i: (0, 0)),
        pl.BlockSpec((1, 1), lambda i: (0, 0)),
    ],
    out_specs=pl.BlockSpec((_BN, 1), lambda i: (i, 0)),
    out_shape=jax.ShapeDtypeStruct((_N, 1), jnp.float32),
)


def kernel(x, edge_index, W1l, b1l, W1r, W2l, b2l, W2r, Wlin1, blin1, Wlin2,
           blin2):
    npad = _EP - _E
    src2d = jnp.concatenate(
        [edge_index[0], jnp.zeros((npad,), jnp.int32)]).reshape(_CR, _K)
    dst2d = jnp.concatenate(
        [edge_index[1], jnp.full((npad,), _TRASH, jnp.int32)]).reshape(_CR, _K)
    xL = x[:, :128]
    xR = x[:, 128:]
    zrow = jnp.zeros((_RT, 128), jnp.float32)
    ones = jnp.ones((_K, 128), jnp.float32)

    cntc, = _count(dst2d, zrow, ones)
    aL, = _agg(xL, src2d, dst2d, zrow)
    aR, = _agg(xR, src2d, dst2d, zrow)
    h1L, h1R, h1 = _dense1(aL, aR, cntc, x, W1l, b1l.reshape(1, _H), W1r)
    a2L, = _agg(h1L, src2d, dst2d, zrow)
    a2R, = _agg(h1R, src2d, dst2d, zrow)
    lg = _dense2(a2L, a2R, cntc, h1, W2l, b2l.reshape(1, _H), W2r,
                 Wlin1, blin1.reshape(1, _H), Wlin2, blin2.reshape(1, 1))
    return lg[:, 0]
